# baseline (speedup 1.0000x reference)
import functools

import jax
import jax.numpy as jnp
from jax import lax
from jax.experimental import pallas as pl
from jax.experimental.pallas import tpu as pltpu
from jax.experimental.pallas import tpu_sc as plsc

N = 10000
NPAD = 10240
HF = 64
E = 160000
FC = 128
G = 64
L = 3
EPS = 1e-5
LOG2 = 0.6931471805599453

TE = 2000
NSTEPS = E // TE

NC = 2
NS = 16
NW = NC * NS
GCH = 128


def _sigmoid(v):
    return 1.0 / (1.0 + jnp.exp(-v))


def _silu(v):
    return v * _sigmoid(v)


def _init_body(x_ref, w_ref, b_ref, o_ref):
    o_ref[...] = (jnp.dot(x_ref[...], w_ref[...],
                          preferred_element_type=jnp.float32) + b_ref[0, :])


def _node_init(x, w, b):
    return pl.pallas_call(
        _init_body,
        out_shape=jax.ShapeDtypeStruct((N, FC), jnp.float32),
    )(x, w, b.reshape(1, FC))


def _edge_feat_body(d_ref, c_ref, g_ref, w_ref, b_ref, o_ref):
    d = d_ref[...]
    diff = d - c_ref[...]
    r = jnp.exp(-g_ref[0, 0] * diff * diff).astype(jnp.bfloat16)
    h = jnp.dot(r, w_ref[...], preferred_element_type=jnp.float32) + b_ref[0, :]
    o_ref[...] = _silu(h).astype(jnp.bfloat16)


def _edge_feat(edge_attr, centers, gamma, w, b):
    return pl.pallas_call(
        _edge_feat_body,
        grid=(NSTEPS,),
        in_specs=[
            pl.BlockSpec((TE, 1), lambda i: (i, 0)),
            pl.BlockSpec((1, FC), lambda i: (0, 0)),
            pl.BlockSpec((1, 1), lambda i: (0, 0)),
            pl.BlockSpec((FC, FC), lambda i: (0, 0)),
            pl.BlockSpec((1, FC), lambda i: (0, 0)),
        ],
        out_specs=pl.BlockSpec((TE, FC), lambda i: (i, 0)),
        out_shape=jax.ShapeDtypeStruct((E, FC), jnp.bfloat16),
    )(edge_attr.reshape(E, 1), centers, gamma, w, b.reshape(1, FC))


def _edge_mlp_body(xi_ref, xj_ref, ef_ref, w1a_ref, w1b_ref, w1c_ref, b1_ref,
                   w2f_ref, b2f_ref, w2m_ref, b2m_ref,
                   hf_ref, hm_ref, st_ref):
    xi = xi_ref[...].astype(jnp.bfloat16)
    xj = xj_ref[...].astype(jnp.bfloat16)
    z = (jnp.dot(xi, w1a_ref[...], preferred_element_type=jnp.float32)
         + jnp.dot(xj, w1b_ref[...], preferred_element_type=jnp.float32)
         + jnp.dot(ef_ref[...], w1c_ref[...], preferred_element_type=jnp.float32)
         + b1_ref[0, :])
    s = _silu(z).astype(jnp.bfloat16)
    hf = (jnp.dot(s[:, :FC], w2f_ref[...], preferred_element_type=jnp.float32)
          + b2f_ref[0, :])
    hm = (jnp.dot(s[:, FC:], w2m_ref[...], preferred_element_type=jnp.float32)
          + b2m_ref[0, :])
    hf_ref[...] = hf.astype(jnp.bfloat16)
    hm_ref[...] = hm.astype(jnp.bfloat16)

    @pl.when(pl.program_id(0) == 0)
    def _():
        st_ref[...] = jnp.zeros((8, FC), jnp.float32)

    st_ref[0:1, :] += jnp.sum(hf, axis=0)[None, :]
    st_ref[1:2, :] += jnp.sum(hf * hf, axis=0)[None, :]


def _edge_mlp(gat, ef, w1a, w1b, w1c, b1, w2f, b2f, w2m, b2m):
    full = lambda shape: pl.BlockSpec(shape, lambda i: (0, 0))
    return pl.pallas_call(
        _edge_mlp_body,
        grid=(NSTEPS,),
        in_specs=[
            pl.BlockSpec((TE, FC), lambda i: (i, 0)),
            pl.BlockSpec((TE, FC), lambda i: (NSTEPS + i, 0)),
            pl.BlockSpec((TE, FC), lambda i: (i, 0)),
            full((FC, 2 * FC)), full((FC, 2 * FC)), full((FC, 2 * FC)),
            full((1, 2 * FC)),
            full((FC, FC)), full((1, FC)),
            full((FC, FC)), full((1, FC)),
        ],
        out_specs=[
            pl.BlockSpec((TE, FC), lambda i: (i, 0)),
            pl.BlockSpec((TE, FC), lambda i: (i, 0)),
            pl.BlockSpec((8, FC), lambda i: (0, 0)),
        ],
        out_shape=[
            jax.ShapeDtypeStruct((E, FC), jnp.bfloat16),
            jax.ShapeDtypeStruct((E, FC), jnp.bfloat16),
            jax.ShapeDtypeStruct((8, FC), jnp.float32),
        ],
    )(gat, gat, ef, w1a, w1b, w1c, b1, w2f, b2f, w2m, b2m)


def _gate_body(hf_ref, hm_ref, st_ref, g_ref, b_ref, m_ref):
    ssum = st_ref[0, :]
    ssq = st_ref[1, :]
    mu = ssum * (1.0 / E)
    var = ssq * (1.0 / E) - mu * mu
    rstd = lax.rsqrt(var + EPS)
    hf = hf_ref[...].astype(jnp.float32)
    hfn = (hf - mu) * (rstd * g_ref[0, :]) + b_ref[0, :]
    m_ref[...] = _sigmoid(hfn) * hm_ref[...].astype(jnp.float32)


def _gate(hf, hm, st, g, b):
    full = lambda shape: pl.BlockSpec(shape, lambda i: (0, 0))
    return pl.pallas_call(
        _gate_body,
        grid=(NSTEPS,),
        in_specs=[
            pl.BlockSpec((TE, FC), lambda i: (i, 0)),
            pl.BlockSpec((TE, FC), lambda i: (i, 0)),
            full((8, FC)), full((1, FC)), full((1, FC)),
        ],
        out_specs=pl.BlockSpec((TE, FC), lambda i: (i, 0)),
        out_shape=jax.ShapeDtypeStruct((E, FC), jnp.float32),
    )(hf, hm, st, g.reshape(1, FC), b.reshape(1, FC))


def _node_update_body(n_ref, a_ref, g_ref, b_ref, o_ref):
    agg = a_ref[:N, :] + a_ref[NPAD:NPAD + N, :]
    mu = jnp.mean(agg, axis=0)
    var = jnp.mean((agg - mu) ** 2, axis=0)
    bn = (agg - mu) * (lax.rsqrt(var + EPS) * g_ref[0, :]) + b_ref[0, :]
    o_ref[...] = jnp.maximum(n_ref[...] + bn, 0.0)


def _node_update(node, aggp, g, b):
    return pl.pallas_call(
        _node_update_body,
        out_shape=jax.ShapeDtypeStruct((N, FC), jnp.float32),
    )(node, aggp, g.reshape(1, FC), b.reshape(1, FC))


def _readout_body(n_ref, bt_ref, fw_ref, fb_ref, ow_ref, ob_ref, o_ref):
    bt = bt_ref[...]
    gi = lax.broadcasted_iota(jnp.int32, (G, N), 0)
    ind = (gi == bt).astype(jnp.float32)
    seg = jnp.dot(ind, n_ref[...], preferred_element_type=jnp.float32)
    cnt = jnp.sum(ind, axis=1)
    mean = seg / jnp.maximum(cnt, 1.0)[:, None]
    h = (jnp.dot(mean, fw_ref[...], preferred_element_type=jnp.float32)
         + fb_ref[0, :])
    feat = jnp.maximum(h, 0.0) + jnp.log1p(jnp.exp(-jnp.abs(h))) - LOG2
    o_ref[...] = jnp.sum(feat * ow_ref[...], axis=1) + ob_ref[0, 0]


def _readout(node, batch, fcw, fcb, outw, outb):
    return pl.pallas_call(
        _readout_body,
        out_shape=jax.ShapeDtypeStruct((G,), jnp.float32),
    )(node, batch.reshape(1, N), fcw, fcb.reshape(1, FC),
      outw.reshape(1, FC), outb.reshape(1, 1))


def _sc_gather(node, idx2e):
    nper = (2 * E) // NW
    nfull = nper // GCH
    tail = nper - nfull * GCH
    mesh = plsc.VectorSubcoreMesh(core_axis_name="c", subcore_axis_name="s")

    @functools.partial(
        pl.kernel,
        out_type=jax.ShapeDtypeStruct((2 * E, FC), jnp.float32),
        mesh=mesh,
        scratch_types=[
            pltpu.VMEM((GCH,), jnp.int32),
            pltpu.VMEM((GCH, FC), jnp.float32),
        ],
    )
    def k(node_hbm, idx_hbm, out_hbm, idx_v, rows_v):
        wid = lax.axis_index("s") * NC + lax.axis_index("c")
        base = wid * nper

        def chunk(off, nrows):
            off = pl.multiple_of(off, 8)
            pltpu.sync_copy(idx_hbm.at[pl.ds(off, nrows)],
                            idx_v.at[pl.ds(0, nrows)])
            pltpu.sync_copy(node_hbm.at[idx_v.at[pl.ds(0, nrows)]],
                            rows_v.at[pl.ds(0, nrows)])
            pltpu.sync_copy(rows_v.at[pl.ds(0, nrows)],
                            out_hbm.at[pl.ds(off, nrows)])

        def body(i, carry):
            chunk(base + i * GCH, GCH)
            return carry

        lax.fori_loop(0, nfull, body, 0)
        chunk(base + nfull * GCH, tail)

    return k(node, idx2e)


def _sc_scatter(m, dst, zrows):
    nper = E // NW
    nfull = nper // GCH
    tail = nper - nfull * GCH
    rpt = NPAD // NS
    mesh = plsc.VectorSubcoreMesh(core_axis_name="c", subcore_axis_name="s")

    @functools.partial(
        pl.kernel,
        out_type=jax.ShapeDtypeStruct((2 * NPAD, FC), jnp.float32),
        mesh=mesh,
        scratch_types=[
            pltpu.VMEM((GCH,), jnp.int32),
            pltpu.VMEM((GCH, FC), jnp.float32),
            pltpu.VMEM_SHARED((NPAD, FC), jnp.float32),
        ],
    )
    def k(m_hbm, dst_hbm, z_hbm, out_hbm, idx_v, buf_v, acc_sh):
        c = lax.axis_index("c")
        s = lax.axis_index("s")
        wid = s * NC + c
        r0 = s * rpt

        pltpu.sync_copy(z_hbm, buf_v)
        for j in range(rpt // GCH):
            pltpu.sync_copy(buf_v, acc_sh.at[pl.ds(r0 + j * GCH, GCH)])
        plsc.subcore_barrier()

        base = wid * nper

        def chunk(off, nrows):
            off = pl.multiple_of(off, 8)
            pltpu.sync_copy(dst_hbm.at[pl.ds(off, nrows)],
                            idx_v.at[pl.ds(0, nrows)])
            pltpu.sync_copy(m_hbm.at[pl.ds(off, nrows)],
                            buf_v.at[pl.ds(0, nrows)])
            pltpu.sync_copy(buf_v.at[pl.ds(0, nrows)],
                            acc_sh.at[idx_v.at[pl.ds(0, nrows)]], add=True)

        def body(i, carry):
            chunk(base + i * GCH, GCH)
            return carry

        lax.fori_loop(0, nfull, body, 0)
        chunk(base + nfull * GCH, tail)
        plsc.subcore_barrier()

        for j in range(rpt // GCH):
            pltpu.sync_copy(acc_sh.at[pl.ds(r0 + j * GCH, GCH)], buf_v)
            pltpu.sync_copy(buf_v,
                            out_hbm.at[pl.ds(c * NPAD + r0 + j * GCH, GCH)])

    return k(m, dst, zrows)


def kernel(x, edge_attr, params, edge_index, batch):
    p = params
    src = edge_index[0]
    dst = edge_index[1]
    idx2e = jnp.concatenate([dst, src])
    centers = jnp.linspace(-4.0, 4.0, FC).astype(jnp.float32)
    gamma = (1.0 / (centers[1] - centers[0]) ** 2).reshape(1, 1)
    zrows = jnp.zeros((GCH, FC), jnp.float32)

    node = _node_init(x, p['atom_W'], p['atom_b'])
    ef = _edge_feat(edge_attr, centers.reshape(1, FC), gamma,
                    p['edge_W'].astype(jnp.bfloat16), p['edge_b'])

    for i in range(L):
        w1cat = jnp.concatenate([p['nfW1_%d' % i], p['nlW1_%d' % i]],
                                axis=1).astype(jnp.bfloat16)
        b1cat = jnp.concatenate([p['nfb1_%d' % i], p['nlb1_%d' % i]]
                                ).reshape(1, 2 * FC)
        gat = _sc_gather(node, idx2e)
        hf, hm, st = _edge_mlp(
            gat, ef,
            w1cat[:FC], w1cat[FC:2 * FC], w1cat[2 * FC:], b1cat,
            p['nfW2_%d' % i].astype(jnp.bfloat16),
            p['nfb2_%d' % i].reshape(1, FC),
            p['nlW2_%d' % i].astype(jnp.bfloat16),
            p['nlb2_%d' % i].reshape(1, FC))
        m = _gate(hf, hm, st, p['bnig_%d' % i], p['bnib_%d' % i])
        aggp = _sc_scatter(m, dst, zrows)
        node = _node_update(node, aggp, p['bng_%d' % i], p['bnb_%d' % i])

    return _readout(node, batch, p['fcW'], p['fcb'], p['outW'], p['outb'])


# speedup vs baseline: 1.0078x; 1.0078x over previous
import functools

import jax
import jax.numpy as jnp
from jax import lax
from jax.experimental import pallas as pl
from jax.experimental.pallas import tpu as pltpu
from jax.experimental.pallas import tpu_sc as plsc

N = 10000
NPAD = 10240
HF = 64
E = 160000
FC = 128
G = 64
L = 3
EPS = 1e-5
LOG2 = 0.6931471805599453

TE = 2000
NSTEPS = E // TE
E2 = E // 2
NSH = E2 // TE

NC = 2
NS = 16
NW = NC * NS
GCH = 128


def _sigmoid(v):
    return 0.5 * jnp.tanh(0.5 * v) + 0.5


def _silu(v):
    return v * _sigmoid(v)


def _init_body(x_ref, w_ref, b_ref, o_ref):
    o_ref[:N, :] = (jnp.dot(x_ref[...], w_ref[...],
                            preferred_element_type=jnp.float32) + b_ref[0, :])
    o_ref[N:, :] = jnp.zeros((NPAD - N, FC), jnp.float32)


def _node_init(x, w, b):
    return pl.pallas_call(
        _init_body,
        out_shape=jax.ShapeDtypeStruct((NPAD, FC), jnp.float32),
    )(x, w, b.reshape(1, FC))


def _edge_feat_body(d_ref, c_ref, g_ref, w_ref, b_ref, o_ref):
    d = d_ref[...]
    diff = d - c_ref[...]
    r = jnp.exp(-g_ref[0, 0] * diff * diff).astype(jnp.bfloat16)
    h = jnp.dot(r, w_ref[...], preferred_element_type=jnp.float32) + b_ref[0, :]
    o_ref[...] = _silu(h).astype(jnp.bfloat16)


def _edge_feat(edge_attr, centers, gamma, w, b):
    return pl.pallas_call(
        _edge_feat_body,
        grid=(NSTEPS,),
        in_specs=[
            pl.BlockSpec((TE, 1), lambda i: (i, 0)),
            pl.BlockSpec((1, FC), lambda i: (0, 0)),
            pl.BlockSpec((1, 1), lambda i: (0, 0)),
            pl.BlockSpec((FC, FC), lambda i: (0, 0)),
            pl.BlockSpec((1, FC), lambda i: (0, 0)),
        ],
        out_specs=pl.BlockSpec((TE, FC), lambda i: (i, 0)),
        out_shape=jax.ShapeDtypeStruct((E, FC), jnp.bfloat16),
    )(edge_attr.reshape(E, 1), centers, gamma, w, b.reshape(1, FC))


def _edge_mlp_body(xi_ref, xj_ref, ef_ref, w1a_ref, w1b_ref, w1c_ref, b1_ref,
                   w2f_ref, b2f_ref, w2m_ref, b2m_ref,
                   hf_ref, hm_ref, st_ref):
    xi = xi_ref[...].astype(jnp.bfloat16)
    xj = xj_ref[...].astype(jnp.bfloat16)
    z = (jnp.dot(xi, w1a_ref[...], preferred_element_type=jnp.float32)
         + jnp.dot(xj, w1b_ref[...], preferred_element_type=jnp.float32)
         + jnp.dot(ef_ref[...], w1c_ref[...], preferred_element_type=jnp.float32)
         + b1_ref[0, :])
    s = _silu(z).astype(jnp.bfloat16)
    hf = (jnp.dot(s[:, :FC], w2f_ref[...], preferred_element_type=jnp.float32)
          + b2f_ref[0, :])
    hm = (jnp.dot(s[:, FC:], w2m_ref[...], preferred_element_type=jnp.float32)
          + b2m_ref[0, :])
    hf_ref[...] = hf.astype(jnp.bfloat16)
    hm_ref[...] = hm.astype(jnp.bfloat16)

    @pl.when(pl.program_id(0) == 0)
    def _():
        st_ref[...] = jnp.zeros((8, FC), jnp.float32)

    st_ref[0:1, :] += jnp.sum(hf, axis=0)[None, :]
    st_ref[1:2, :] += jnp.sum(hf * hf, axis=0)[None, :]


def _edge_mlp(gat, ef, half, w1a, w1b, w1c, b1, w2f, b2f, w2m, b2m):
    full = lambda shape: pl.BlockSpec(shape, lambda i: (0, 0))
    return pl.pallas_call(
        _edge_mlp_body,
        grid=(NSH,),
        in_specs=[
            pl.BlockSpec((TE, FC), lambda i, h=half: (2 * h * NSH + i, 0)),
            pl.BlockSpec((TE, FC), lambda i, h=half: ((2 * h + 1) * NSH + i, 0)),
            pl.BlockSpec((TE, FC), lambda i, h=half: (h * NSH + i, 0)),
            full((FC, 2 * FC)), full((FC, 2 * FC)), full((FC, 2 * FC)),
            full((1, 2 * FC)),
            full((FC, FC)), full((1, FC)),
            full((FC, FC)), full((1, FC)),
        ],
        out_specs=[
            pl.BlockSpec((TE, FC), lambda i: (i, 0)),
            pl.BlockSpec((TE, FC), lambda i: (i, 0)),
            pl.BlockSpec((8, FC), lambda i: (0, 0)),
        ],
        out_shape=[
            jax.ShapeDtypeStruct((E2, FC), jnp.bfloat16),
            jax.ShapeDtypeStruct((E2, FC), jnp.bfloat16),
            jax.ShapeDtypeStruct((8, FC), jnp.float32),
        ],
    )(gat, gat, ef, w1a, w1b, w1c, b1, w2f, b2f, w2m, b2m)


def _gate_body(hf_ref, hm_ref, st_ref, st2_ref, g_ref, b_ref, m_ref):
    ssum = st_ref[0, :] + st2_ref[0, :]
    ssq = st_ref[1, :] + st2_ref[1, :]
    mu = ssum * (1.0 / E)
    var = ssq * (1.0 / E) - mu * mu
    rstd = lax.rsqrt(var + EPS)
    hf = hf_ref[...].astype(jnp.float32)
    hfn = (hf - mu) * (rstd * g_ref[0, :]) + b_ref[0, :]
    m_ref[...] = _sigmoid(hfn) * hm_ref[...].astype(jnp.float32)


def _gate(hf, hm, st, st2, g, b):
    full = lambda shape: pl.BlockSpec(shape, lambda i: (0, 0))
    return pl.pallas_call(
        _gate_body,
        grid=(NSH,),
        in_specs=[
            pl.BlockSpec((TE, FC), lambda i: (i, 0)),
            pl.BlockSpec((TE, FC), lambda i: (i, 0)),
            full((8, FC)), full((8, FC)), full((1, FC)), full((1, FC)),
        ],
        out_specs=pl.BlockSpec((TE, FC), lambda i: (i, 0)),
        out_shape=jax.ShapeDtypeStruct((E2, FC), jnp.float32),
    )(hf, hm, st, st2, g.reshape(1, FC), b.reshape(1, FC))


def _node_update_body(n_ref, a_ref, a2_ref, g_ref, b_ref, o_ref):
    agg = (a_ref[:N, :] + a_ref[NPAD:NPAD + N, :]
           + a2_ref[:N, :] + a2_ref[NPAD:NPAD + N, :])
    mu = jnp.mean(agg, axis=0)
    var = jnp.mean((agg - mu) ** 2, axis=0)
    bn = (agg - mu) * (lax.rsqrt(var + EPS) * g_ref[0, :]) + b_ref[0, :]
    o_ref[:N, :] = jnp.maximum(n_ref[:N, :] + bn, 0.0)
    o_ref[N:, :] = jnp.zeros((NPAD - N, FC), jnp.float32)


def _node_update(node, aggp, aggp2, g, b):
    return pl.pallas_call(
        _node_update_body,
        out_shape=jax.ShapeDtypeStruct((NPAD, FC), jnp.float32),
    )(node, aggp, aggp2, g.reshape(1, FC), b.reshape(1, FC))


def _readout_body(n_ref, bt_ref, fw_ref, fb_ref, ow_ref, ob_ref, o_ref):
    bt = bt_ref[...]
    gi = lax.broadcasted_iota(jnp.int32, (G, N), 0)
    ind = (gi == bt).astype(jnp.float32)
    seg = jnp.dot(ind, n_ref[:N, :], preferred_element_type=jnp.float32)
    cnt = jnp.sum(ind, axis=1)
    mean = seg / jnp.maximum(cnt, 1.0)[:, None]
    h = (jnp.dot(mean, fw_ref[...], preferred_element_type=jnp.float32)
         + fb_ref[0, :])
    feat = jnp.maximum(h, 0.0) + jnp.log1p(jnp.exp(-jnp.abs(h))) - LOG2
    o_ref[...] = jnp.sum(feat * ow_ref[...], axis=1) + ob_ref[0, 0]


def _readout(node, batch, fcw, fcb, outw, outb):
    return pl.pallas_call(
        _readout_body,
        out_shape=jax.ShapeDtypeStruct((G,), jnp.float32),
    )(node, batch.reshape(1, N), fcw, fcb.reshape(1, FC),
      outw.reshape(1, FC), outb.reshape(1, 1))


def _sc_gather(node, idx):
    tot = 4 * E2
    nper = tot // NW
    nfull = nper // GCH
    tail = nper - nfull * GCH
    rpt = NPAD // NS
    mesh = plsc.VectorSubcoreMesh(core_axis_name="c", subcore_axis_name="s")

    @functools.partial(
        pl.kernel,
        out_type=jax.ShapeDtypeStruct((tot, FC), jnp.float32),
        mesh=mesh,
        scratch_types=[
            pltpu.VMEM((GCH,), jnp.int32),
            pltpu.VMEM((GCH, FC), jnp.float32),
            pltpu.VMEM_SHARED((NPAD, FC), jnp.float32),
        ],
    )
    def k(node_hbm, idx_hbm, out_hbm, idx_v, rows_v, tab_sh):
        c = lax.axis_index("c")
        s = lax.axis_index("s")
        wid = s * NC + c
        r0 = s * rpt
        for j in range(rpt // GCH):
            pltpu.sync_copy(node_hbm.at[pl.ds(r0 + j * GCH, GCH)],
                            tab_sh.at[pl.ds(r0 + j * GCH, GCH)])
        plsc.subcore_barrier()

        base = wid * nper

        def chunk(off, nrows):
            off = pl.multiple_of(off, 8)
            pltpu.sync_copy(idx_hbm.at[pl.ds(off, nrows)],
                            idx_v.at[pl.ds(0, nrows)])
            pltpu.sync_copy(node_hbm.at[idx_v.at[pl.ds(0, nrows)]],
                            rows_v.at[pl.ds(0, nrows)])
            pltpu.sync_copy(rows_v.at[pl.ds(0, nrows)],
                            out_hbm.at[pl.ds(off, nrows)])

        def body(i, carry):
            chunk(base + i * GCH, GCH)
            return carry

        lax.fori_loop(0, nfull, body, 0)
        chunk(base + nfull * GCH, tail)

    return k(node, idx)


def _sc_scatter(m, dst, zrows):
    nper = 2496
    nfull = nper // GCH
    tail = nper - nfull * GCH
    rpt = NPAD // NS
    mesh = plsc.VectorSubcoreMesh(core_axis_name="c", subcore_axis_name="s")

    @functools.partial(
        pl.kernel,
        out_type=jax.ShapeDtypeStruct((2 * NPAD, FC), jnp.float32),
        mesh=mesh,
        scratch_types=[
            pltpu.VMEM((GCH,), jnp.int32),
            pltpu.VMEM((GCH, FC), jnp.float32),
            pltpu.VMEM_SHARED((NPAD, FC), jnp.float32),
        ],
    )
    def k(m_hbm, dst_hbm, z_hbm, out_hbm, idx_v, buf_v, acc_sh):
        c = lax.axis_index("c")
        s = lax.axis_index("s")
        wid = s * NC + c
        r0 = s * rpt

        pltpu.sync_copy(z_hbm, buf_v)
        for j in range(rpt // GCH):
            pltpu.sync_copy(buf_v, acc_sh.at[pl.ds(r0 + j * GCH, GCH)])
        plsc.subcore_barrier()

        base = wid * nper
        nf_w = nfull + (wid == NW - 1).astype(jnp.int32)

        def chunk(off, nrows):
            off = pl.multiple_of(off, 8)
            pltpu.sync_copy(dst_hbm.at[pl.ds(off, nrows)],
                            idx_v.at[pl.ds(0, nrows)])
            pltpu.sync_copy(m_hbm.at[pl.ds(off, nrows)],
                            buf_v.at[pl.ds(0, nrows)])
            pltpu.sync_copy(buf_v.at[pl.ds(0, nrows)],
                            acc_sh.at[idx_v.at[pl.ds(0, nrows)]], add=True)

        def body(i, carry):
            chunk(base + i * GCH, GCH)
            return carry

        lax.fori_loop(0, nf_w, body, 0, unroll=False)
        chunk(base + nf_w * GCH, tail)
        plsc.subcore_barrier()

        for j in range(rpt // GCH):
            pltpu.sync_copy(acc_sh.at[pl.ds(r0 + j * GCH, GCH)], buf_v)
            pltpu.sync_copy(buf_v,
                            out_hbm.at[pl.ds(c * NPAD + r0 + j * GCH, GCH)])

    return k(m, dst, zrows)


def kernel(x, edge_attr, params, edge_index, batch):
    p = params
    src = edge_index[0]
    dst = edge_index[1]
    idx_all = jnp.concatenate([dst[:E2], src[:E2], dst[E2:], src[E2:]])
    dst_h = [dst[:E2], dst[E2:]]
    centers = jnp.linspace(-4.0, 4.0, FC).astype(jnp.float32)
    gamma = (1.0 / (centers[1] - centers[0]) ** 2).reshape(1, 1)
    zrows = jnp.zeros((GCH, FC), jnp.float32)

    node = _node_init(x, p['atom_W'], p['atom_b'])
    ef = _edge_feat(edge_attr, centers.reshape(1, FC), gamma,
                    p['edge_W'].astype(jnp.bfloat16), p['edge_b'])

    for i in range(L):
        w1cat = jnp.concatenate([p['nfW1_%d' % i], p['nlW1_%d' % i]],
                                axis=1).astype(jnp.bfloat16)
        b1cat = jnp.concatenate([p['nfb1_%d' % i], p['nlb1_%d' % i]]
                                ).reshape(1, 2 * FC)
        mlp_w = (w1cat[:FC], w1cat[FC:2 * FC], w1cat[2 * FC:], b1cat,
                 p['nfW2_%d' % i].astype(jnp.bfloat16),
                 p['nfb2_%d' % i].reshape(1, FC),
                 p['nlW2_%d' % i].astype(jnp.bfloat16),
                 p['nlb2_%d' % i].reshape(1, FC))
        gat = _sc_gather(node, idx_all)
        hf0, hm0, st0 = _edge_mlp(gat, ef, 0, *mlp_w)
        hf1, hm1, st1 = _edge_mlp(gat, ef, 1, *mlp_w)
        m0 = _gate(hf0, hm0, st0, st1, p['bnig_%d' % i], p['bnib_%d' % i])
        m1 = _gate(hf1, hm1, st0, st1, p['bnig_%d' % i], p['bnib_%d' % i])
        agg0 = _sc_scatter(m0, dst_h[0], zrows)
        agg1 = _sc_scatter(m1, dst_h[1], zrows)
        node = _node_update(node, agg0, agg1,
                            p['bng_%d' % i], p['bnb_%d' % i])

    return _readout(node, batch, p['fcW'], p['fcb'], p['outW'], p['outb'])


# speedup vs baseline: 1.1314x; 1.1227x over previous
import functools

import jax
import jax.numpy as jnp
from jax import lax
from jax.experimental import pallas as pl
from jax.experimental.pallas import tpu as pltpu
from jax.experimental.pallas import tpu_sc as plsc

N = 10000
NPAD = 10240
HF = 64
E = 160000
FC = 128
G = 64
L = 3
EPS = 1e-5
LOG2 = 0.6931471805599453

TE = 2000
NSTEPS = E // TE
E2 = E // 2
NSH = E2 // TE

NC = 2
NS = 16
NW = NC * NS
GCH = 128


def _sigmoid(v):
    return 0.5 * jnp.tanh(0.5 * v) + 0.5


def _silu(v):
    return v * _sigmoid(v)


def _init_body(x_ref, w_ref, b_ref, o_ref):
    o_ref[:N, :] = (jnp.dot(x_ref[...], w_ref[...],
                            preferred_element_type=jnp.float32) + b_ref[0, :])
    o_ref[N:, :] = jnp.zeros((NPAD - N, FC), jnp.float32)


def _node_init(x, w, b):
    return pl.pallas_call(
        _init_body,
        out_shape=jax.ShapeDtypeStruct((NPAD, FC), jnp.float32),
    )(x, w, b.reshape(1, FC))


def _edge_feat_body(d_ref, c_ref, g_ref, w_ref, b_ref, o_ref):
    d = d_ref[...]
    diff = d - c_ref[...]
    r = jnp.exp(-g_ref[0, 0] * diff * diff).astype(jnp.bfloat16)
    h = jnp.dot(r, w_ref[...], preferred_element_type=jnp.float32) + b_ref[0, :]
    o_ref[...] = _silu(h).astype(jnp.bfloat16)


def _edge_feat(edge_attr, centers, gamma, w, b):
    return pl.pallas_call(
        _edge_feat_body,
        grid=(NSTEPS,),
        in_specs=[
            pl.BlockSpec((TE, 1), lambda i: (i, 0)),
            pl.BlockSpec((1, FC), lambda i: (0, 0)),
            pl.BlockSpec((1, 1), lambda i: (0, 0)),
            pl.BlockSpec((FC, FC), lambda i: (0, 0)),
            pl.BlockSpec((1, FC), lambda i: (0, 0)),
        ],
        out_specs=pl.BlockSpec((TE, FC), lambda i: (i, 0)),
        out_shape=jax.ShapeDtypeStruct((E, FC), jnp.bfloat16),
    )(edge_attr.reshape(E, 1), centers, gamma, w, b.reshape(1, FC))


def _edge_mlp_body(xi_ref, xj_ref, ef_ref, w1a_ref, w1b_ref, w1c_ref, b1_ref,
                   w2f_ref, b2f_ref, w2m_ref, b2m_ref,
                   hf_ref, hm_ref, st_ref):
    xi = xi_ref[...].astype(jnp.bfloat16)
    xj = xj_ref[...].astype(jnp.bfloat16)
    z = (jnp.dot(xi, w1a_ref[...], preferred_element_type=jnp.float32)
         + jnp.dot(xj, w1b_ref[...], preferred_element_type=jnp.float32)
         + jnp.dot(ef_ref[...], w1c_ref[...], preferred_element_type=jnp.float32)
         + b1_ref[0, :])
    s = _silu(z).astype(jnp.bfloat16)
    hf = (jnp.dot(s[:, :FC], w2f_ref[...], preferred_element_type=jnp.float32)
          + b2f_ref[0, :])
    hm = (jnp.dot(s[:, FC:], w2m_ref[...], preferred_element_type=jnp.float32)
          + b2m_ref[0, :])
    hf_ref[...] = hf
    hm_ref[...] = hm

    @pl.when(pl.program_id(0) == 0)
    def _():
        st_ref[...] = jnp.zeros((8, FC), jnp.float32)

    st_ref[0:1, :] += jnp.sum(hf, axis=0)[None, :]
    st_ref[1:2, :] += jnp.sum(hf * hf, axis=0)[None, :]


def _edge_mlp(gat, ef, half, w1a, w1b, w1c, b1, w2f, b2f, w2m, b2m):
    full = lambda shape: pl.BlockSpec(shape, lambda i: (0, 0))
    return pl.pallas_call(
        _edge_mlp_body,
        grid=(NSH,),
        in_specs=[
            pl.BlockSpec((TE, FC), lambda i, h=half: (2 * h * NSH + i, 0)),
            pl.BlockSpec((TE, FC), lambda i, h=half: ((2 * h + 1) * NSH + i, 0)),
            pl.BlockSpec((TE, FC), lambda i, h=half: (h * NSH + i, 0)),
            full((FC, 2 * FC)), full((FC, 2 * FC)), full((FC, 2 * FC)),
            full((1, 2 * FC)),
            full((FC, FC)), full((1, FC)),
            full((FC, FC)), full((1, FC)),
        ],
        out_specs=[
            pl.BlockSpec((TE, FC), lambda i: (i, 0)),
            pl.BlockSpec((TE, FC), lambda i: (i, 0)),
            pl.BlockSpec((8, FC), lambda i: (0, 0)),
        ],
        out_shape=[
            jax.ShapeDtypeStruct((E2, FC), jnp.float32),
            jax.ShapeDtypeStruct((E2, FC), jnp.float32),
            jax.ShapeDtypeStruct((8, FC), jnp.float32),
        ],
    )(gat, gat, ef, w1a, w1b, w1c, b1, w2f, b2f, w2m, b2m)


def _gate_body(hf_ref, hm_ref, st_ref, st2_ref, g_ref, b_ref, m_ref):
    ssum = st_ref[0, :] + st2_ref[0, :]
    ssq = st_ref[1, :] + st2_ref[1, :]
    mu = ssum * (1.0 / E)
    var = ssq * (1.0 / E) - mu * mu
    rstd = lax.rsqrt(var + EPS)
    hf = hf_ref[...].astype(jnp.float32)
    hfn = (hf - mu) * (rstd * g_ref[0, :]) + b_ref[0, :]
    m_ref[...] = _sigmoid(hfn) * hm_ref[...].astype(jnp.float32)


def _gate(hf, hm, st, st2, g, b):
    full = lambda shape: pl.BlockSpec(shape, lambda i: (0, 0))
    return pl.pallas_call(
        _gate_body,
        grid=(NSH,),
        in_specs=[
            pl.BlockSpec((TE, FC), lambda i: (i, 0)),
            pl.BlockSpec((TE, FC), lambda i: (i, 0)),
            full((8, FC)), full((8, FC)), full((1, FC)), full((1, FC)),
        ],
        out_specs=pl.BlockSpec((TE, FC), lambda i: (i, 0)),
        out_shape=jax.ShapeDtypeStruct((E2, FC), jnp.float32),
    )(hf, hm, st, st2, g.reshape(1, FC), b.reshape(1, FC))


def _node_update_body(n_ref, a_ref, a2_ref, g_ref, b_ref, o_ref):
    agg = (a_ref[:N, :] + a_ref[NPAD:NPAD + N, :]
           + a2_ref[:N, :] + a2_ref[NPAD:NPAD + N, :])
    mu = jnp.mean(agg, axis=0)
    var = jnp.mean((agg - mu) ** 2, axis=0)
    bn = (agg - mu) * (lax.rsqrt(var + EPS) * g_ref[0, :]) + b_ref[0, :]
    o_ref[:N, :] = jnp.maximum(n_ref[:N, :] + bn, 0.0)
    o_ref[N:, :] = jnp.zeros((NPAD - N, FC), jnp.float32)


def _node_update(node, aggp, aggp2, g, b):
    return pl.pallas_call(
        _node_update_body,
        out_shape=jax.ShapeDtypeStruct((NPAD, FC), jnp.float32),
    )(node, aggp, aggp2, g.reshape(1, FC), b.reshape(1, FC))


def _readout_body(n_ref, bt_ref, fw_ref, fb_ref, ow_ref, ob_ref, o_ref):
    bt = bt_ref[...]
    gi = lax.broadcasted_iota(jnp.int32, (G, N), 0)
    ind = (gi == bt).astype(jnp.float32)
    seg = jnp.dot(ind, n_ref[:N, :], preferred_element_type=jnp.float32)
    cnt = jnp.sum(ind, axis=1)
    mean = seg / jnp.maximum(cnt, 1.0)[:, None]
    h = (jnp.dot(mean, fw_ref[...], preferred_element_type=jnp.float32)
         + fb_ref[0, :])
    feat = jnp.maximum(h, 0.0) + jnp.log1p(jnp.exp(-jnp.abs(h))) - LOG2
    o_ref[...] = jnp.sum(feat * ow_ref[...], axis=1) + ob_ref[0, 0]


def _readout(node, batch, fcw, fcb, outw, outb):
    return pl.pallas_call(
        _readout_body,
        out_shape=jax.ShapeDtypeStruct((G,), jnp.float32),
    )(node, batch.reshape(1, N), fcw, fcb.reshape(1, FC),
      outw.reshape(1, FC), outb.reshape(1, 1))


def _sc_gather(node, idx):
    tot = 4 * E2
    nper = tot // NW
    nfull = nper // GCH
    tail = nper - nfull * GCH
    mesh = plsc.VectorSubcoreMesh(core_axis_name="c", subcore_axis_name="s")

    @functools.partial(
        pl.kernel,
        out_type=jax.ShapeDtypeStruct((tot, FC), jnp.float32),
        mesh=mesh,
        scratch_types=[
            pltpu.VMEM((nper,), jnp.int32),
            pltpu.VMEM((GCH, FC), jnp.float32),
            pltpu.VMEM((GCH, FC), jnp.float32),
            pltpu.SemaphoreType.DMA,
            pltpu.SemaphoreType.DMA,
        ],
    )
    def k(node_hbm, idx_hbm, out_hbm, idx_v, r0_v, r1_v, s0_sem, s1_sem):
        wid = lax.axis_index("s") * NC + lax.axis_index("c")
        base = wid * nper
        pltpu.sync_copy(idx_hbm.at[pl.ds(pl.multiple_of(base, 8), nper)],
                        idx_v)

        bufs = (r0_v, r1_v)
        sems = (s0_sem, s1_sem)

        def g_start(i, b):
            pltpu.async_copy(node_hbm.at[idx_v.at[pl.ds(i * GCH, GCH)]],
                             bufs[b], sems[b])

        def w_start(i, b):
            off = pl.multiple_of(base + i * GCH, 8)
            pltpu.async_copy(bufs[b], out_hbm.at[pl.ds(off, GCH)], sems[b])

        def drain(b):
            pltpu.make_async_copy(node_hbm.at[pl.ds(0, GCH)],
                                  bufs[b], sems[b]).wait()

        g_start(0, 0)
        g_start(1, 1)

        def body(j, carry):
            for b in range(2):
                i = j * 2 + b
                drain(b)
                w_start(i, b)
                drain(b)
                g_start(i + 2, b)
            return carry

        nep = 2 + nfull % 2
        lax.fori_loop(0, (nfull - nep) // 2, body, 0, unroll=False)
        for i in range(nfull - nep, nfull):
            b = i % 2
            drain(b)
            w_start(i, b)
            drain(b)
            if i + 2 < nfull:
                g_start(i + 2, b)

        off = pl.multiple_of(base + nfull * GCH, 8)
        pltpu.sync_copy(node_hbm.at[idx_v.at[pl.ds(nfull * GCH, tail)]],
                        r0_v.at[pl.ds(0, tail)])
        pltpu.sync_copy(r0_v.at[pl.ds(0, tail)],
                        out_hbm.at[pl.ds(off, tail)])

    return k(node, idx)


def _sc_scatter(m, dst, zrows):
    nper = 2496
    nfull = nper // GCH
    tail = nper - nfull * GCH
    rpt = NPAD // NS
    mesh = plsc.VectorSubcoreMesh(core_axis_name="c", subcore_axis_name="s")

    @functools.partial(
        pl.kernel,
        out_type=jax.ShapeDtypeStruct((2 * NPAD, FC), jnp.float32),
        mesh=mesh,
        scratch_types=[
            pltpu.VMEM((GCH,), jnp.int32),
            pltpu.VMEM((GCH, FC), jnp.float32),
            pltpu.VMEM_SHARED((NPAD, FC), jnp.float32),
        ],
    )
    def k(m_hbm, dst_hbm, z_hbm, out_hbm, idx_v, buf_v, acc_sh):
        c = lax.axis_index("c")
        s = lax.axis_index("s")
        wid = s * NC + c
        r0 = s * rpt

        pltpu.sync_copy(z_hbm, buf_v)
        for j in range(rpt // GCH):
            pltpu.sync_copy(buf_v, acc_sh.at[pl.ds(r0 + j * GCH, GCH)])
        plsc.subcore_barrier()

        base = wid * nper
        nf_w = nfull + (wid == NW - 1).astype(jnp.int32)

        def chunk(off, nrows):
            off = pl.multiple_of(off, 8)
            pltpu.sync_copy(dst_hbm.at[pl.ds(off, nrows)],
                            idx_v.at[pl.ds(0, nrows)])
            pltpu.sync_copy(m_hbm.at[pl.ds(off, nrows)],
                            buf_v.at[pl.ds(0, nrows)])
            pltpu.sync_copy(buf_v.at[pl.ds(0, nrows)],
                            acc_sh.at[idx_v.at[pl.ds(0, nrows)]], add=True)

        def body(i, carry):
            chunk(base + i * GCH, GCH)
            return carry

        lax.fori_loop(0, nf_w, body, 0, unroll=False)
        chunk(base + nf_w * GCH, tail)
        plsc.subcore_barrier()

        for j in range(rpt // GCH):
            pltpu.sync_copy(acc_sh.at[pl.ds(r0 + j * GCH, GCH)], buf_v)
            pltpu.sync_copy(buf_v,
                            out_hbm.at[pl.ds(c * NPAD + r0 + j * GCH, GCH)])

    return k(m, dst, zrows)


def kernel(x, edge_attr, params, edge_index, batch):
    p = params
    src = edge_index[0]
    dst = edge_index[1]
    idx_all = jnp.concatenate([dst[:E2], src[:E2], dst[E2:], src[E2:]])
    dst_h = [dst[:E2], dst[E2:]]
    centers = jnp.linspace(-4.0, 4.0, FC).astype(jnp.float32)
    gamma = (1.0 / (centers[1] - centers[0]) ** 2).reshape(1, 1)
    zrows = jnp.zeros((GCH, FC), jnp.float32)

    node = _node_init(x, p['atom_W'], p['atom_b'])
    ef = _edge_feat(edge_attr, centers.reshape(1, FC), gamma,
                    p['edge_W'].astype(jnp.bfloat16), p['edge_b'])

    for i in range(L):
        w1cat = jnp.concatenate([p['nfW1_%d' % i], p['nlW1_%d' % i]],
                                axis=1).astype(jnp.bfloat16)
        b1cat = jnp.concatenate([p['nfb1_%d' % i], p['nlb1_%d' % i]]
                                ).reshape(1, 2 * FC)
        mlp_w = (w1cat[:FC], w1cat[FC:2 * FC], w1cat[2 * FC:], b1cat,
                 p['nfW2_%d' % i].astype(jnp.bfloat16),
                 p['nfb2_%d' % i].reshape(1, FC),
                 p['nlW2_%d' % i].astype(jnp.bfloat16),
                 p['nlb2_%d' % i].reshape(1, FC))
        gat = _sc_gather(node, idx_all)
        hf0, hm0, st0 = _edge_mlp(gat, ef, 0, *mlp_w)
        hf1, hm1, st1 = _edge_mlp(gat, ef, 1, *mlp_w)
        m0 = _gate(hf0, hm0, st0, st1, p['bnig_%d' % i], p['bnib_%d' % i])
        m1 = _gate(hf1, hm1, st0, st1, p['bnig_%d' % i], p['bnib_%d' % i])
        agg0 = _sc_scatter(m0, dst_h[0], zrows)
        agg1 = _sc_scatter(m1, dst_h[1], zrows)
        node = _node_update(node, agg0, agg1,
                            p['bng_%d' % i], p['bnb_%d' % i])

    return _readout(node, batch, p['fcW'], p['fcb'], p['outW'], p['outb'])
